# Initial kernel scaffold; baseline (speedup 1.0000x reference)
import jax
import jax.numpy as jnp
from jax.experimental import pallas as pl
from jax.experimental.pallas import tpu as pltpu

S = 8192
DIM = 2048
N_EXPERT = 64
K = 2
BLK = 512
GRID = S // BLK


def _gate_kernel(x_ref, w_ref, b_ref, ids_ref, vals_ref, aux_ref,
                 psum_ref, cnt_ref):
    step = pl.program_id(0)

    logits = jnp.dot(x_ref[...], w_ref[...],
                     preferred_element_type=jnp.float32) + b_ref[...]

    m = jnp.max(logits, axis=1, keepdims=True)
    e = jnp.exp(logits - m)
    s = jnp.sum(e, axis=1, keepdims=True)
    prob = e / s

    lane = jax.lax.broadcasted_iota(jnp.int32, prob.shape, 1)
    i1 = jnp.argmax(prob, axis=1).astype(jnp.int32)
    v1 = jnp.max(prob, axis=1)
    masked = jnp.where(lane == i1[:, None], -1.0, prob)
    i2 = jnp.argmax(masked, axis=1).astype(jnp.int32)
    v2 = jnp.max(masked, axis=1)

    ids_ref[...] = jnp.stack([i1, i2], axis=1)
    vals_ref[...] = jnp.stack([v1, v2], axis=1)

    one_hot = (lane == i1[:, None]).astype(jnp.float32)
    blk_cnt = jnp.sum(one_hot, axis=0, keepdims=True)
    blk_psum = jnp.sum(prob, axis=0, keepdims=True)

    @pl.when(step == 0)
    def _init():
        psum_ref[...] = blk_psum
        cnt_ref[...] = blk_cnt

    @pl.when(step != 0)
    def _acc():
        psum_ref[...] += blk_psum
        cnt_ref[...] += blk_cnt

    @pl.when(step == GRID - 1)
    def _finalize():
        aux_ref[0, 0] = (float(N_EXPERT) / (S * S)) * jnp.sum(
            psum_ref[...] * cnt_ref[...])


@jax.jit
def kernel(x, W, b):
    ids, vals, aux = pl.pallas_call(
        _gate_kernel,
        grid=(GRID,),
        in_specs=[
            pl.BlockSpec((BLK, DIM), lambda i: (i, 0)),
            pl.BlockSpec((DIM, N_EXPERT), lambda i: (0, 0)),
            pl.BlockSpec((1, N_EXPERT), lambda i: (0, 0)),
        ],
        out_specs=[
            pl.BlockSpec((BLK, K), lambda i: (i, 0)),
            pl.BlockSpec((BLK, K), lambda i: (i, 0)),
            pl.BlockSpec((1, 1), lambda i: (0, 0)),
        ],
        out_shape=[
            jax.ShapeDtypeStruct((S, K), jnp.int32),
            jax.ShapeDtypeStruct((S, K), jnp.float32),
            jax.ShapeDtypeStruct((1, 1), jnp.float32),
        ],
        scratch_shapes=[
            pltpu.VMEM((1, N_EXPERT), jnp.float32),
            pltpu.VMEM((1, N_EXPERT), jnp.float32),
        ],
    )(x, W, b.reshape(1, N_EXPERT))
    return ids, vals, aux[0, 0]


# speedup vs baseline: 2.6794x; 2.6794x over previous
import jax
import jax.numpy as jnp
from jax.experimental import pallas as pl
from jax.experimental.pallas import tpu as pltpu

S = 8192
DIM = 2048
N_EXPERT = 64
K = 2
BLK = 512
GRID = S // BLK


def _gate_kernel(x_ref, w_ref, b_ref, ids_ref, vals_ref, aux_ref,
                 psum_ref, cnt_ref):
    step = pl.program_id(0)

    logits = jnp.dot(x_ref[...], w_ref[...],
                     preferred_element_type=jnp.float32) + b_ref[...]

    m = jnp.max(logits, axis=1, keepdims=True)
    e = jnp.exp(logits - m)
    s = jnp.sum(e, axis=1, keepdims=True)
    prob = e / s

    lane = jax.lax.broadcasted_iota(jnp.int32, prob.shape, 1)
    i1 = jnp.argmax(prob, axis=1).astype(jnp.int32)
    v1 = jnp.max(prob, axis=1)
    masked = jnp.where(lane == i1[:, None], -1.0, prob)
    i2 = jnp.argmax(masked, axis=1).astype(jnp.int32)
    v2 = jnp.max(masked, axis=1)

    ids_ref[...] = jnp.stack([i1, i2], axis=1)
    vals_ref[...] = jnp.stack([v1, v2], axis=1)

    one_hot = (lane == i1[:, None]).astype(jnp.float32)
    blk_cnt = jnp.sum(one_hot, axis=0, keepdims=True)
    blk_psum = jnp.sum(prob, axis=0, keepdims=True)

    @pl.when(step == 0)
    def _init():
        psum_ref[...] = blk_psum
        cnt_ref[...] = blk_cnt

    @pl.when(step != 0)
    def _acc():
        psum_ref[...] += blk_psum
        cnt_ref[...] += blk_cnt

    @pl.when(step == GRID - 1)
    def _finalize():
        aux_ref[...] = (float(N_EXPERT) / (S * S)) * jnp.sum(
            psum_ref[...] * cnt_ref[...], axis=1, keepdims=True)


@jax.jit
def kernel(x, W, b):
    ids, vals, aux = pl.pallas_call(
        _gate_kernel,
        grid=(GRID,),
        in_specs=[
            pl.BlockSpec((BLK, DIM), lambda i: (i, 0)),
            pl.BlockSpec((DIM, N_EXPERT), lambda i: (0, 0)),
            pl.BlockSpec((1, N_EXPERT), lambda i: (0, 0)),
        ],
        out_specs=[
            pl.BlockSpec((BLK, K), lambda i: (i, 0)),
            pl.BlockSpec((BLK, K), lambda i: (i, 0)),
            pl.BlockSpec((1, 1), lambda i: (0, 0)),
        ],
        out_shape=[
            jax.ShapeDtypeStruct((S, K), jnp.int32),
            jax.ShapeDtypeStruct((S, K), jnp.float32),
            jax.ShapeDtypeStruct((1, 1), jnp.float32),
        ],
        scratch_shapes=[
            pltpu.VMEM((1, N_EXPERT), jnp.float32),
            pltpu.VMEM((1, N_EXPERT), jnp.float32),
        ],
    )(x, W, b.reshape(1, N_EXPERT))
    return ids, vals, aux[0, 0]


# speedup vs baseline: 2.9949x; 1.1177x over previous
import jax
import jax.numpy as jnp
from jax.experimental import pallas as pl
from jax.experimental.pallas import tpu as pltpu

S = 8192
DIM = 2048
N_EXPERT = 64
K = 2
BLK = 1024
GRID = S // BLK


def _gate_kernel(x_ref, w_ref, b_ref, ids_ref, vals_ref, aux_ref,
                 psum_ref, cnt_ref):
    step = pl.program_id(0)

    logits = jnp.dot(x_ref[...], w_ref[...],
                     preferred_element_type=jnp.float32) + b_ref[...]

    m = jnp.max(logits, axis=1, keepdims=True)
    e = jnp.exp(logits - m)
    s = jnp.sum(e, axis=1, keepdims=True)
    prob = e / s

    lane = jax.lax.broadcasted_iota(jnp.int32, prob.shape, 1)
    i1 = jnp.argmax(prob, axis=1).astype(jnp.int32)
    v1 = jnp.max(prob, axis=1)
    masked = jnp.where(lane == i1[:, None], -1.0, prob)
    i2 = jnp.argmax(masked, axis=1).astype(jnp.int32)
    v2 = jnp.max(masked, axis=1)

    ids_ref[...] = jnp.stack([i1, i2], axis=1)
    vals_ref[...] = jnp.stack([v1, v2], axis=1)

    one_hot = (lane == i1[:, None]).astype(jnp.float32)
    blk_cnt = jnp.sum(one_hot, axis=0, keepdims=True)
    blk_psum = jnp.sum(prob, axis=0, keepdims=True)

    @pl.when(step == 0)
    def _init():
        psum_ref[...] = blk_psum
        cnt_ref[...] = blk_cnt

    @pl.when(step != 0)
    def _acc():
        psum_ref[...] += blk_psum
        cnt_ref[...] += blk_cnt

    @pl.when(step == GRID - 1)
    def _finalize():
        aux_ref[...] = (float(N_EXPERT) / (S * S)) * jnp.sum(
            psum_ref[...] * cnt_ref[...], axis=1, keepdims=True)


@jax.jit
def kernel(x, W, b):
    ids, vals, aux = pl.pallas_call(
        _gate_kernel,
        grid=(GRID,),
        in_specs=[
            pl.BlockSpec((BLK, DIM), lambda i: (i, 0)),
            pl.BlockSpec((DIM, N_EXPERT), lambda i: (0, 0)),
            pl.BlockSpec((1, N_EXPERT), lambda i: (0, 0)),
        ],
        out_specs=[
            pl.BlockSpec((BLK, K), lambda i: (i, 0)),
            pl.BlockSpec((BLK, K), lambda i: (i, 0)),
            pl.BlockSpec((1, 1), lambda i: (0, 0)),
        ],
        out_shape=[
            jax.ShapeDtypeStruct((S, K), jnp.int32),
            jax.ShapeDtypeStruct((S, K), jnp.float32),
            jax.ShapeDtypeStruct((1, 1), jnp.float32),
        ],
        scratch_shapes=[
            pltpu.VMEM((1, N_EXPERT), jnp.float32),
            pltpu.VMEM((1, N_EXPERT), jnp.float32),
        ],
    )(x, W, b.reshape(1, N_EXPERT))
    return ids, vals, aux[0, 0]


# speedup vs baseline: 3.0011x; 1.0021x over previous
import jax
import jax.numpy as jnp
from jax.experimental import pallas as pl
from jax.experimental.pallas import tpu as pltpu

S = 8192
DIM = 2048
N_EXPERT = 64
K = 2
BLK = 2048
GRID = S // BLK


def _gate_kernel(x_ref, w_ref, b_ref, ids_ref, vals_ref, aux_ref,
                 psum_ref, cnt_ref):
    step = pl.program_id(0)

    logits = jnp.dot(x_ref[...], w_ref[...],
                     preferred_element_type=jnp.float32) + b_ref[...]

    m = jnp.max(logits, axis=1, keepdims=True)
    e = jnp.exp(logits - m)
    s = jnp.sum(e, axis=1, keepdims=True)
    prob = e / s

    lane = jax.lax.broadcasted_iota(jnp.int32, prob.shape, 1)
    i1 = jnp.argmax(prob, axis=1).astype(jnp.int32)
    v1 = jnp.max(prob, axis=1)
    masked = jnp.where(lane == i1[:, None], -1.0, prob)
    i2 = jnp.argmax(masked, axis=1).astype(jnp.int32)
    v2 = jnp.max(masked, axis=1)

    ids_ref[...] = jnp.stack([i1, i2], axis=1)
    vals_ref[...] = jnp.stack([v1, v2], axis=1)

    one_hot = (lane == i1[:, None]).astype(jnp.float32)
    blk_cnt = jnp.sum(one_hot, axis=0, keepdims=True)
    blk_psum = jnp.sum(prob, axis=0, keepdims=True)

    @pl.when(step == 0)
    def _init():
        psum_ref[...] = blk_psum
        cnt_ref[...] = blk_cnt

    @pl.when(step != 0)
    def _acc():
        psum_ref[...] += blk_psum
        cnt_ref[...] += blk_cnt

    @pl.when(step == GRID - 1)
    def _finalize():
        aux_ref[...] = (float(N_EXPERT) / (S * S)) * jnp.sum(
            psum_ref[...] * cnt_ref[...], axis=1, keepdims=True)


@jax.jit
def kernel(x, W, b):
    ids, vals, aux = pl.pallas_call(
        _gate_kernel,
        grid=(GRID,),
        in_specs=[
            pl.BlockSpec((BLK, DIM), lambda i: (i, 0)),
            pl.BlockSpec((DIM, N_EXPERT), lambda i: (0, 0)),
            pl.BlockSpec((1, N_EXPERT), lambda i: (0, 0)),
        ],
        out_specs=[
            pl.BlockSpec((BLK, K), lambda i: (i, 0)),
            pl.BlockSpec((BLK, K), lambda i: (i, 0)),
            pl.BlockSpec((1, 1), lambda i: (0, 0)),
        ],
        out_shape=[
            jax.ShapeDtypeStruct((S, K), jnp.int32),
            jax.ShapeDtypeStruct((S, K), jnp.float32),
            jax.ShapeDtypeStruct((1, 1), jnp.float32),
        ],
        scratch_shapes=[
            pltpu.VMEM((1, N_EXPERT), jnp.float32),
            pltpu.VMEM((1, N_EXPERT), jnp.float32),
        ],
    )(x, W, b.reshape(1, N_EXPERT))
    return ids, vals, aux[0, 0]


# speedup vs baseline: 3.0572x; 1.0187x over previous
import jax
import jax.numpy as jnp
from jax.experimental import pallas as pl
from jax.experimental.pallas import tpu as pltpu

S = 8192
DIM = 2048
N_EXPERT = 64
K = 2
NSPLIT = 2
SUB = 512
BLK = NSPLIT * SUB
GRID = S // BLK


def _gate_kernel(*refs):
    x_refs = refs[:NSPLIT]
    w_ref, b_ref, ids_ref, vals_ref, aux_ref, psum_ref, cnt_ref = refs[NSPLIT:]
    step = pl.program_id(0)

    logits = jnp.concatenate(
        [jnp.dot(xr[...], w_ref[...], preferred_element_type=jnp.float32)
         for xr in x_refs], axis=0) + b_ref[...]

    m = jnp.max(logits, axis=1, keepdims=True)
    e = jnp.exp(logits - m)
    s = jnp.sum(e, axis=1, keepdims=True)
    prob = e / s

    lane = jax.lax.broadcasted_iota(jnp.int32, prob.shape, 1)
    i1 = jnp.argmax(prob, axis=1).astype(jnp.int32)
    v1 = jnp.max(prob, axis=1)
    masked = jnp.where(lane == i1[:, None], -1.0, prob)
    i2 = jnp.argmax(masked, axis=1).astype(jnp.int32)
    v2 = jnp.max(masked, axis=1)

    ids_ref[...] = jnp.stack([i1, i2], axis=1)
    vals_ref[...] = jnp.stack([v1, v2], axis=1)

    one_hot = (lane == i1[:, None]).astype(jnp.float32)
    blk_cnt = jnp.sum(one_hot, axis=0, keepdims=True)
    blk_psum = jnp.sum(prob, axis=0, keepdims=True)

    @pl.when(step == 0)
    def _init():
        psum_ref[...] = blk_psum
        cnt_ref[...] = blk_cnt

    @pl.when(step != 0)
    def _acc():
        psum_ref[...] += blk_psum
        cnt_ref[...] += blk_cnt

    @pl.when(step == GRID - 1)
    def _finalize():
        aux_ref[...] = (float(N_EXPERT) / (S * S)) * jnp.sum(
            psum_ref[...] * cnt_ref[...], axis=1, keepdims=True)


def _x_spec(j):
    return pl.BlockSpec((SUB, DIM), lambda i, j=j: (NSPLIT * i + j, 0))


@jax.jit
def kernel(x, W, b):
    ids, vals, aux = pl.pallas_call(
        _gate_kernel,
        grid=(GRID,),
        in_specs=[_x_spec(j) for j in range(NSPLIT)] + [
            pl.BlockSpec((DIM, N_EXPERT), lambda i: (0, 0)),
            pl.BlockSpec((1, N_EXPERT), lambda i: (0, 0)),
        ],
        out_specs=[
            pl.BlockSpec((BLK, K), lambda i: (i, 0)),
            pl.BlockSpec((BLK, K), lambda i: (i, 0)),
            pl.BlockSpec((1, 1), lambda i: (0, 0)),
        ],
        out_shape=[
            jax.ShapeDtypeStruct((S, K), jnp.int32),
            jax.ShapeDtypeStruct((S, K), jnp.float32),
            jax.ShapeDtypeStruct((1, 1), jnp.float32),
        ],
        scratch_shapes=[
            pltpu.VMEM((1, N_EXPERT), jnp.float32),
            pltpu.VMEM((1, N_EXPERT), jnp.float32),
        ],
    )(*([x] * NSPLIT), W, b.reshape(1, N_EXPERT))
    return ids, vals, aux[0, 0]
